# Initial kernel scaffold; baseline (speedup 1.0000x reference)
import jax
import jax.numpy as jnp
from jax.experimental import pallas as pl


def kernel(input, Adj, Wg1, bg1, Wg2, bg2, Wt1, bt1, Wt2, bt2):
    raise NotImplementedError("write your pallas kernel here")


# speedup vs baseline: 18.6729x; 18.6729x over previous
import jax
import jax.numpy as jnp
from jax.experimental import pallas as pl
from jax.experimental.pallas import tpu as pltpu

_N = 4096
_F = 128
_H = 128
_C = 64
_K = 50
_EPS = 1e-12
_BR = 512
_NB = _N // _BR
_PREC = jax.lax.Precision.DEFAULT


def _dinv(d):
    return jnp.where(d > 0, 1.0 / jnp.sqrt(jnp.maximum(d, _EPS)), 0.0)


def _dot(a, b):
    return jnp.dot(a, b, preferred_element_type=jnp.float32, precision=_PREC)


def _sim_dot(a, b):
    hh = _F // 2
    s1 = jax.lax.dot_general(a[:, :hh], b[:, :hh], (((1,), (1,)), ((), ())),
                             preferred_element_type=jnp.float32, precision=_PREC)
    s2 = jax.lax.dot_general(a[:, hh:], b[:, hh:], (((1,), (1,)), ((), ())),
                             preferred_element_type=jnp.float32, precision=_PREC)
    return s1 + s2


def _key(x):
    b = jax.lax.bitcast_convert_type(x, jnp.int32)
    return b ^ ((b >> 31) & jnp.int32(0x7FFFFFFF))


def _k1(x_ref, adj_ref, wg1_ref, wt1_ref, d_ref, drow_ref, p1_ref, pt1_ref):
    adj = adj_ref[...]
    d_ref[...] = jnp.sum(adj, axis=1, keepdims=True)
    cs = jnp.sum(adj, axis=0, keepdims=True)

    @pl.when(pl.program_id(0) == 0)
    def _():
        drow_ref[...] = cs

    @pl.when(pl.program_id(0) != 0)
    def _():
        drow_ref[...] += cs

    xb = x_ref[...]
    p1_ref[...] = _dot(xb, wg1_ref[...])
    pt1_ref[...] = _dot(xb, wt1_ref[...])


def _k2(adj_ref, p1_ref, drow_ref, db_ref, b1_ref, w2_ref, na_ref, z_ref):
    na = (_dinv(db_ref[...]) * adj_ref[...]) * _dinv(drow_ref[...])
    na_ref[...] = na
    h = jnp.maximum(_dot(na, p1_ref[...]) + b1_ref[...], 0.0)
    z_ref[...] = _dot(h, w2_ref[...])


def _k3(na_ref, z_ref, b2_ref, emb_ref):
    e = _dot(na_ref[...], z_ref[...]) + b2_ref[...]
    rn = jnp.sqrt(jnp.sum(e * e, axis=1, keepdims=True))
    emb_ref[...] = e / jnp.maximum(rn, _EPS)


def _k4(embb_ref, emb_ref, tcol_ref, keys_ref):
    eb = embb_ref[...]
    for j in range(_NB):
        ch = emb_ref[j * _BR:(j + 1) * _BR, :]
        keys_ref[:, j * _BR:(j + 1) * _BR] = _key(_sim_dot(eb, ch))
    keys = keys_ref[...]
    lo = jnp.min(keys, axis=1, keepdims=True) - 1
    hi = jnp.max(keys, axis=1, keepdims=True)

    def body(_, carry):
        lo, hi = carry
        mid = lo + ((hi - lo + 1) >> 1)
        c = jnp.sum((keys > mid).astype(jnp.int32), axis=1, keepdims=True)
        ge = c >= _K
        return jnp.where(ge, mid, lo), jnp.where(ge, hi, mid)

    lo, hi = jax.lax.fori_loop(0, 32, body, (lo, hi))
    tcol_ref[...] = hi


def _k5(embi_ref, embj_ref, tki_ref, tkj_ref, adj_ref, na_ref, af_ref, d2_ref):
    s = _sim_dot(embi_ref[...], embj_ref[...])
    ks = _key(s)
    mr = (ks >= tki_ref[...]).astype(jnp.float32)
    mc = (ks >= tkj_ref[...]).astype(jnp.float32)
    na = 0.5 * s * (mr + mc)
    af = na + adj_ref[...]
    na_ref[...] = na
    af_ref[...] = af
    rs = jnp.sum(af, axis=1, keepdims=True)

    @pl.when(pl.program_id(1) == 0)
    def _():
        d2_ref[...] = rs

    @pl.when(pl.program_id(1) != 0)
    def _():
        d2_ref[...] += rs


def _k6(af_ref, pt1_ref, d2_ref, db_ref, b1_ref, w2_ref, o_ref):
    yt = _dinv(d2_ref[...]) * pt1_ref[...]
    acc = _dot(af_ref[...], yt)
    db = _dinv(db_ref[...])
    ht = jnp.maximum(db * acc + b1_ref[...], 0.0)
    o_ref[...] = db * _dot(ht, w2_ref[...])


def _k7(af_ref, z_ref, db_ref, b2_ref, o_ref):
    o_ref[...] = _dinv(db_ref[...]) * _dot(af_ref[...], z_ref[...]) + b2_ref[...]


def _blk(shape, imap):
    return pl.BlockSpec(shape, imap)


def kernel(input, Adj, Wg1, bg1, Wg2, bg2, Wt1, bt1, Wt2, bt2):
    x = input
    f32 = jnp.float32
    bg1r = bg1.reshape(1, _H)
    bg2r = bg2.reshape(1, _H)
    bt1r = bt1.reshape(1, _H)
    bt2r = bt2.reshape(1, _C)

    d, drow, P1, Pt1 = pl.pallas_call(
        _k1,
        grid=(_NB,),
        in_specs=[
            _blk((_BR, _F), lambda i: (i, 0)),
            _blk((_BR, _N), lambda i: (i, 0)),
            _blk((_F, _H), lambda i: (0, 0)),
            _blk((_F, _H), lambda i: (0, 0)),
        ],
        out_specs=[
            _blk((_BR, 1), lambda i: (i, 0)),
            _blk((1, _N), lambda i: (0, 0)),
            _blk((_BR, _H), lambda i: (i, 0)),
            _blk((_BR, _H), lambda i: (i, 0)),
        ],
        out_shape=[
            jax.ShapeDtypeStruct((_N, 1), f32),
            jax.ShapeDtypeStruct((1, _N), f32),
            jax.ShapeDtypeStruct((_N, _H), f32),
            jax.ShapeDtypeStruct((_N, _H), f32),
        ],
    )(x, Adj, Wg1, Wt1)

    nA, Z = pl.pallas_call(
        _k2,
        grid=(_NB,),
        in_specs=[
            _blk((_BR, _N), lambda i: (i, 0)),
            _blk((_N, _H), lambda i: (0, 0)),
            _blk((1, _N), lambda i: (0, 0)),
            _blk((_BR, 1), lambda i: (i, 0)),
            _blk((1, _H), lambda i: (0, 0)),
            _blk((_H, _H), lambda i: (0, 0)),
        ],
        out_specs=[
            _blk((_BR, _N), lambda i: (i, 0)),
            _blk((_BR, _H), lambda i: (i, 0)),
        ],
        out_shape=[
            jax.ShapeDtypeStruct((_N, _N), f32),
            jax.ShapeDtypeStruct((_N, _H), f32),
        ],
    )(Adj, P1, drow, d, bg1r, Wg2)

    emb = pl.pallas_call(
        _k3,
        grid=(_NB,),
        in_specs=[
            _blk((_BR, _N), lambda i: (i, 0)),
            _blk((_N, _H), lambda i: (0, 0)),
            _blk((1, _H), lambda i: (0, 0)),
        ],
        out_specs=_blk((_BR, _H), lambda i: (i, 0)),
        out_shape=jax.ShapeDtypeStruct((_N, _H), f32),
    )(nA, Z, bg2r)

    tcol = pl.pallas_call(
        _k4,
        grid=(_NB,),
        in_specs=[
            _blk((_BR, _F), lambda i: (i, 0)),
            _blk((_N, _F), lambda i: (0, 0)),
        ],
        out_specs=_blk((_BR, 1), lambda i: (i, 0)),
        out_shape=jax.ShapeDtypeStruct((_N, 1), jnp.int32),
        scratch_shapes=[pltpu.VMEM((_BR, _N), jnp.int32)],
    )(emb, emb)

    trow = tcol.reshape(1, _N)

    new_adj, AF, d2 = pl.pallas_call(
        _k5,
        grid=(_NB, _NB),
        in_specs=[
            _blk((_BR, _F), lambda i, j: (i, 0)),
            _blk((_BR, _F), lambda i, j: (j, 0)),
            _blk((_BR, 1), lambda i, j: (i, 0)),
            _blk((1, _BR), lambda i, j: (0, j)),
            _blk((_BR, _BR), lambda i, j: (i, j)),
        ],
        out_specs=[
            _blk((_BR, _BR), lambda i, j: (i, j)),
            _blk((_BR, _BR), lambda i, j: (i, j)),
            _blk((_BR, 1), lambda i, j: (i, 0)),
        ],
        out_shape=[
            jax.ShapeDtypeStruct((_N, _N), f32),
            jax.ShapeDtypeStruct((_N, _N), f32),
            jax.ShapeDtypeStruct((_N, 1), f32),
        ],
    )(emb, emb, tcol, trow, Adj)

    Z2t = pl.pallas_call(
        _k6,
        grid=(_NB,),
        in_specs=[
            _blk((_BR, _N), lambda i: (i, 0)),
            _blk((_N, _H), lambda i: (0, 0)),
            _blk((_N, 1), lambda i: (0, 0)),
            _blk((_BR, 1), lambda i: (i, 0)),
            _blk((1, _H), lambda i: (0, 0)),
            _blk((_H, _C), lambda i: (0, 0)),
        ],
        out_specs=_blk((_BR, _C), lambda i: (i, 0)),
        out_shape=jax.ShapeDtypeStruct((_N, _C), f32),
    )(AF, Pt1, d2, d2, bt1r, Wt2)

    x_out = pl.pallas_call(
        _k7,
        grid=(_NB,),
        in_specs=[
            _blk((_BR, _N), lambda i: (i, 0)),
            _blk((_N, _C), lambda i: (0, 0)),
            _blk((_BR, 1), lambda i: (i, 0)),
            _blk((1, _C), lambda i: (0, 0)),
        ],
        out_specs=_blk((_BR, _C), lambda i: (i, 0)),
        out_shape=jax.ShapeDtypeStruct((_N, _C), f32),
    )(AF, Z2t, d2, bt2r)

    return (x_out, new_adj, AF)


# speedup vs baseline: 20.0505x; 1.0738x over previous
import jax
import jax.numpy as jnp
from jax.experimental import pallas as pl
from jax.experimental.pallas import tpu as pltpu

_N = 4096
_F = 128
_H = 128
_C = 64
_K = 50
_EPS = 1e-12
_BR = 512
_NB = _N // _BR
_PREC = jax.lax.Precision.DEFAULT


def _dinv(d):
    return jnp.where(d > 0, 1.0 / jnp.sqrt(jnp.maximum(d, _EPS)), 0.0)


def _dot(a, b):
    return jnp.dot(a, b, preferred_element_type=jnp.float32, precision=_PREC)


def _sim_dot(a, b):
    hh = _F // 2
    s1 = jax.lax.dot_general(a[:, :hh], b[:, :hh], (((1,), (1,)), ((), ())),
                             preferred_element_type=jnp.float32, precision=_PREC)
    s2 = jax.lax.dot_general(a[:, hh:], b[:, hh:], (((1,), (1,)), ((), ())),
                             preferred_element_type=jnp.float32, precision=_PREC)
    return s1 + s2


def _key(x):
    b = jax.lax.bitcast_convert_type(x, jnp.int32)
    return b ^ ((b >> 31) & jnp.int32(0x7FFFFFFF))


def _k1(x_ref, adj_ref, wg1_ref, wt1_ref, d_ref, drow_ref, p1_ref, pt1_ref,
        a8_ref):
    adj = adj_ref[...]
    a8_ref[...] = adj.astype(jnp.int8)
    d_ref[...] = jnp.sum(adj, axis=1, keepdims=True)
    cs = jnp.sum(adj, axis=0, keepdims=True)

    @pl.when(pl.program_id(0) == 0)
    def _():
        drow_ref[...] = cs

    @pl.when(pl.program_id(0) != 0)
    def _():
        drow_ref[...] += cs

    xb = x_ref[...]
    p1_ref[...] = _dot(xb, wg1_ref[...])
    pt1_ref[...] = _dot(xb, wt1_ref[...])


def _na_tile(a8, db, drow):
    return (_dinv(db) * a8.astype(jnp.float32)) * _dinv(drow)


def _k2(a8_ref, p1_ref, drow_ref, db_ref, b1_ref, w2_ref, z_ref):
    na = _na_tile(a8_ref[...], db_ref[...], drow_ref[...])
    h = jnp.maximum(_dot(na, p1_ref[...]) + b1_ref[...], 0.0)
    z_ref[...] = _dot(h, w2_ref[...])


def _k3(a8_ref, z_ref, drow_ref, db_ref, b2_ref, emb_ref):
    na = _na_tile(a8_ref[...], db_ref[...], drow_ref[...])
    e = _dot(na, z_ref[...]) + b2_ref[...]
    rn = jnp.sqrt(jnp.sum(e * e, axis=1, keepdims=True))
    emb_ref[...] = e / jnp.maximum(rn, _EPS)


def _k4(embb_ref, emb_ref, tcol_ref, keys_ref):
    eb = embb_ref[...]
    for j in range(_NB):
        ch = emb_ref[j * _BR:(j + 1) * _BR, :]
        keys_ref[:, j * _BR:(j + 1) * _BR] = _key(_sim_dot(eb, ch))
    keys = keys_ref[...]
    lo = jnp.min(keys, axis=1, keepdims=True) - 1
    hi = jnp.max(keys, axis=1, keepdims=True)

    def body(_, carry):
        lo, hi = carry
        mid = lo + ((hi - lo + 1) >> 1)
        c = jnp.sum((keys > mid).astype(jnp.int32), axis=1, keepdims=True)
        ge = c >= _K
        return jnp.where(ge, mid, lo), jnp.where(ge, hi, mid)

    lo, hi = jax.lax.fori_loop(0, 32, body, (lo, hi))
    tcol_ref[...] = hi


def _k5(embi_ref, embj_ref, tki_ref, tkj_ref, a8_ref, na_ref, af_ref, d2_ref):
    s = _sim_dot(embi_ref[...], embj_ref[...])
    ks = _key(s)
    mr = (ks >= tki_ref[...]).astype(jnp.float32)
    mc = (ks >= tkj_ref[...]).astype(jnp.float32)
    na = 0.5 * s * (mr + mc)
    af = na + a8_ref[...].astype(jnp.float32)
    na_ref[...] = na
    af_ref[...] = af
    rs = jnp.sum(af, axis=1, keepdims=True)

    @pl.when(pl.program_id(1) == 0)
    def _():
        d2_ref[...] = rs

    @pl.when(pl.program_id(1) != 0)
    def _():
        d2_ref[...] += rs


def _k6(af_ref, pt1_ref, d2_ref, db_ref, b1_ref, w2_ref, o_ref):
    yt = _dinv(d2_ref[...]) * pt1_ref[...]
    acc = _dot(af_ref[...], yt)
    db = _dinv(db_ref[...])
    ht = jnp.maximum(db * acc + b1_ref[...], 0.0)
    o_ref[...] = db * _dot(ht, w2_ref[...])


def _k7(af_ref, z_ref, db_ref, b2_ref, o_ref):
    o_ref[...] = _dinv(db_ref[...]) * _dot(af_ref[...], z_ref[...]) + b2_ref[...]


def _blk(shape, imap):
    return pl.BlockSpec(shape, imap)


def kernel(input, Adj, Wg1, bg1, Wg2, bg2, Wt1, bt1, Wt2, bt2):
    x = input
    f32 = jnp.float32
    bg1r = bg1.reshape(1, _H)
    bg2r = bg2.reshape(1, _H)
    bt1r = bt1.reshape(1, _H)
    bt2r = bt2.reshape(1, _C)

    d, drow, P1, Pt1, A8 = pl.pallas_call(
        _k1,
        grid=(_NB,),
        in_specs=[
            _blk((_BR, _F), lambda i: (i, 0)),
            _blk((_BR, _N), lambda i: (i, 0)),
            _blk((_F, _H), lambda i: (0, 0)),
            _blk((_F, _H), lambda i: (0, 0)),
        ],
        out_specs=[
            _blk((_BR, 1), lambda i: (i, 0)),
            _blk((1, _N), lambda i: (0, 0)),
            _blk((_BR, _H), lambda i: (i, 0)),
            _blk((_BR, _H), lambda i: (i, 0)),
            _blk((_BR, _N), lambda i: (i, 0)),
        ],
        out_shape=[
            jax.ShapeDtypeStruct((_N, 1), f32),
            jax.ShapeDtypeStruct((1, _N), f32),
            jax.ShapeDtypeStruct((_N, _H), f32),
            jax.ShapeDtypeStruct((_N, _H), f32),
            jax.ShapeDtypeStruct((_N, _N), jnp.int8),
        ],
    )(x, Adj, Wg1, Wt1)

    Z = pl.pallas_call(
        _k2,
        grid=(_NB,),
        in_specs=[
            _blk((_BR, _N), lambda i: (i, 0)),
            _blk((_N, _H), lambda i: (0, 0)),
            _blk((1, _N), lambda i: (0, 0)),
            _blk((_BR, 1), lambda i: (i, 0)),
            _blk((1, _H), lambda i: (0, 0)),
            _blk((_H, _H), lambda i: (0, 0)),
        ],
        out_specs=_blk((_BR, _H), lambda i: (i, 0)),
        out_shape=jax.ShapeDtypeStruct((_N, _H), f32),
    )(A8, P1, drow, d, bg1r, Wg2)

    emb = pl.pallas_call(
        _k3,
        grid=(_NB,),
        in_specs=[
            _blk((_BR, _N), lambda i: (i, 0)),
            _blk((_N, _H), lambda i: (0, 0)),
            _blk((1, _N), lambda i: (0, 0)),
            _blk((_BR, 1), lambda i: (i, 0)),
            _blk((1, _H), lambda i: (0, 0)),
        ],
        out_specs=_blk((_BR, _H), lambda i: (i, 0)),
        out_shape=jax.ShapeDtypeStruct((_N, _H), f32),
    )(A8, Z, drow, d, bg2r)

    tcol = pl.pallas_call(
        _k4,
        grid=(_NB,),
        in_specs=[
            _blk((_BR, _F), lambda i: (i, 0)),
            _blk((_N, _F), lambda i: (0, 0)),
        ],
        out_specs=_blk((_BR, 1), lambda i: (i, 0)),
        out_shape=jax.ShapeDtypeStruct((_N, 1), jnp.int32),
        scratch_shapes=[pltpu.VMEM((_BR, _N), jnp.int32)],
    )(emb, emb)

    trow = tcol.reshape(1, _N)

    new_adj, AF, d2 = pl.pallas_call(
        _k5,
        grid=(_NB, _NB),
        in_specs=[
            _blk((_BR, _F), lambda i, j: (i, 0)),
            _blk((_BR, _F), lambda i, j: (j, 0)),
            _blk((_BR, 1), lambda i, j: (i, 0)),
            _blk((1, _BR), lambda i, j: (0, j)),
            _blk((_BR, _BR), lambda i, j: (i, j)),
        ],
        out_specs=[
            _blk((_BR, _BR), lambda i, j: (i, j)),
            _blk((_BR, _BR), lambda i, j: (i, j)),
            _blk((_BR, 1), lambda i, j: (i, 0)),
        ],
        out_shape=[
            jax.ShapeDtypeStruct((_N, _N), f32),
            jax.ShapeDtypeStruct((_N, _N), f32),
            jax.ShapeDtypeStruct((_N, 1), f32),
        ],
    )(emb, emb, tcol, trow, A8)

    Z2t = pl.pallas_call(
        _k6,
        grid=(_NB,),
        in_specs=[
            _blk((_BR, _N), lambda i: (i, 0)),
            _blk((_N, _H), lambda i: (0, 0)),
            _blk((_N, 1), lambda i: (0, 0)),
            _blk((_BR, 1), lambda i: (i, 0)),
            _blk((1, _H), lambda i: (0, 0)),
            _blk((_H, _C), lambda i: (0, 0)),
        ],
        out_specs=_blk((_BR, _C), lambda i: (i, 0)),
        out_shape=jax.ShapeDtypeStruct((_N, _C), f32),
    )(AF, Pt1, d2, d2, bt1r, Wt2)

    x_out = pl.pallas_call(
        _k7,
        grid=(_NB,),
        in_specs=[
            _blk((_BR, _N), lambda i: (i, 0)),
            _blk((_N, _C), lambda i: (0, 0)),
            _blk((_BR, 1), lambda i: (i, 0)),
            _blk((1, _C), lambda i: (0, 0)),
        ],
        out_specs=_blk((_BR, _C), lambda i: (i, 0)),
        out_shape=jax.ShapeDtypeStruct((_N, _C), f32),
    )(AF, Z2t, d2, bt2r)

    return (x_out, new_adj, AF)


# speedup vs baseline: 21.6500x; 1.0798x over previous
import jax
import jax.numpy as jnp
from jax.experimental import pallas as pl
from jax.experimental.pallas import tpu as pltpu

_N = 4096
_F = 128
_H = 128
_C = 64
_K = 50
_EPS = 1e-12
_BR = 512
_NB = _N // _BR
_PREC = jax.lax.Precision.DEFAULT


def _dinv(d):
    return jnp.where(d > 0, 1.0 / jnp.sqrt(jnp.maximum(d, _EPS)), 0.0)


def _dot(a, b):
    return jnp.dot(a, b, preferred_element_type=jnp.float32, precision=_PREC)


def _sim_dot(a, b):
    hh = _F // 2
    s1 = jax.lax.dot_general(a[:, :hh], b[:, :hh], (((1,), (1,)), ((), ())),
                             preferred_element_type=jnp.float32, precision=_PREC)
    s2 = jax.lax.dot_general(a[:, hh:], b[:, hh:], (((1,), (1,)), ((), ())),
                             preferred_element_type=jnp.float32, precision=_PREC)
    return s1 + s2


def _key(x):
    b = jax.lax.bitcast_convert_type(x, jnp.int32)
    return b ^ ((b >> 31) & jnp.int32(0x7FFFFFFF))


def _k1(x_ref, adj_ref, wg1_ref, wt1_ref, d_ref, drow_ref, p1_ref, pt1_ref,
        a8_ref):
    adj = adj_ref[...]
    a8_ref[...] = adj.astype(jnp.int8)
    d_ref[...] = jnp.sum(adj, axis=1, keepdims=True)
    cs = jnp.sum(adj, axis=0, keepdims=True)

    @pl.when(pl.program_id(0) == 0)
    def _():
        drow_ref[...] = cs

    @pl.when(pl.program_id(0) != 0)
    def _():
        drow_ref[...] += cs

    xb = x_ref[...]
    p1_ref[...] = _dot(xb, wg1_ref[...])
    pt1_ref[...] = _dot(xb, wt1_ref[...])


def _na_tile(a8, db, drow):
    return (_dinv(db) * a8.astype(jnp.float32)) * _dinv(drow)


def _k2(a8_ref, p1_ref, drow_ref, db_ref, b1_ref, w2_ref, z_ref):
    na = _na_tile(a8_ref[...], db_ref[...], drow_ref[...])
    h = jnp.maximum(_dot(na, p1_ref[...]) + b1_ref[...], 0.0)
    z_ref[...] = _dot(h, w2_ref[...])


def _k3(a8_ref, z_ref, drow_ref, db_ref, b2_ref, emb_ref):
    na = _na_tile(a8_ref[...], db_ref[...], drow_ref[...])
    e = _dot(na, z_ref[...]) + b2_ref[...]
    rn = jnp.sqrt(jnp.sum(e * e, axis=1, keepdims=True))
    emb_ref[...] = e / jnp.maximum(rn, _EPS)


def _k4(embb_ref, emb_ref, tcol_ref, keys_ref):
    eb = embb_ref[...]
    for j in range(_NB):
        ch = emb_ref[j * _BR:(j + 1) * _BR, :]
        keys_ref[:, j * _BR:(j + 1) * _BR] = _key(_sim_dot(eb, ch))
    keys = keys_ref[...]
    lo = jnp.min(keys, axis=1, keepdims=True) - 1
    hi = jnp.max(keys, axis=1, keepdims=True)

    def body(_, carry):
        lo, hi = carry
        mid = lo + ((hi - lo + 1) >> 1)
        c = jnp.sum((keys > mid).astype(jnp.int32), axis=1, keepdims=True)
        ge = c >= _K
        return jnp.where(ge, mid, lo), jnp.where(ge, hi, mid)

    lo, hi = jax.lax.fori_loop(0, 31, body, (lo, hi), unroll=4)
    tcol_ref[...] = hi


def _k5(embi_ref, embj_ref, tki_ref, tkj_ref, a8_ref, na_ref, af_ref, d2_ref):
    s = _sim_dot(embi_ref[...], embj_ref[...])
    ks = _key(s)
    mr = (ks >= tki_ref[...]).astype(jnp.float32)
    mc = (ks >= tkj_ref[...]).astype(jnp.float32)
    na = 0.5 * s * (mr + mc)
    af = na + a8_ref[...].astype(jnp.float32)
    na_ref[...] = na
    af_ref[...] = af
    rs = jnp.sum(af, axis=1, keepdims=True)

    @pl.when(pl.program_id(1) == 0)
    def _():
        d2_ref[...] = rs

    @pl.when(pl.program_id(1) != 0)
    def _():
        d2_ref[...] += rs


def _k6(af_ref, pt1_ref, d2_ref, db_ref, b1_ref, w2_ref, o_ref):
    yt = _dinv(d2_ref[...]) * pt1_ref[...]
    acc = _dot(af_ref[...], yt)
    db = _dinv(db_ref[...])
    ht = jnp.maximum(db * acc + b1_ref[...], 0.0)
    o_ref[...] = db * _dot(ht, w2_ref[...])


def _k7(af_ref, z_ref, db_ref, b2_ref, o_ref):
    o_ref[...] = _dinv(db_ref[...]) * _dot(af_ref[...], z_ref[...]) + b2_ref[...]


def _blk(shape, imap):
    return pl.BlockSpec(shape, imap)


def kernel(input, Adj, Wg1, bg1, Wg2, bg2, Wt1, bt1, Wt2, bt2):
    x = input
    f32 = jnp.float32
    bg1r = bg1.reshape(1, _H)
    bg2r = bg2.reshape(1, _H)
    bt1r = bt1.reshape(1, _H)
    bt2r = bt2.reshape(1, _C)

    d, drow, P1, Pt1, A8 = pl.pallas_call(
        _k1,
        grid=(_NB,),
        in_specs=[
            _blk((_BR, _F), lambda i: (i, 0)),
            _blk((_BR, _N), lambda i: (i, 0)),
            _blk((_F, _H), lambda i: (0, 0)),
            _blk((_F, _H), lambda i: (0, 0)),
        ],
        out_specs=[
            _blk((_BR, 1), lambda i: (i, 0)),
            _blk((1, _N), lambda i: (0, 0)),
            _blk((_BR, _H), lambda i: (i, 0)),
            _blk((_BR, _H), lambda i: (i, 0)),
            _blk((_BR, _N), lambda i: (i, 0)),
        ],
        out_shape=[
            jax.ShapeDtypeStruct((_N, 1), f32),
            jax.ShapeDtypeStruct((1, _N), f32),
            jax.ShapeDtypeStruct((_N, _H), f32),
            jax.ShapeDtypeStruct((_N, _H), f32),
            jax.ShapeDtypeStruct((_N, _N), jnp.int8),
        ],
    )(x, Adj, Wg1, Wt1)

    Z = pl.pallas_call(
        _k2,
        grid=(_NB,),
        in_specs=[
            _blk((_BR, _N), lambda i: (i, 0)),
            _blk((_N, _H), lambda i: (0, 0)),
            _blk((1, _N), lambda i: (0, 0)),
            _blk((_BR, 1), lambda i: (i, 0)),
            _blk((1, _H), lambda i: (0, 0)),
            _blk((_H, _H), lambda i: (0, 0)),
        ],
        out_specs=_blk((_BR, _H), lambda i: (i, 0)),
        out_shape=jax.ShapeDtypeStruct((_N, _H), f32),
    )(A8, P1, drow, d, bg1r, Wg2)

    emb = pl.pallas_call(
        _k3,
        grid=(_NB,),
        in_specs=[
            _blk((_BR, _N), lambda i: (i, 0)),
            _blk((_N, _H), lambda i: (0, 0)),
            _blk((1, _N), lambda i: (0, 0)),
            _blk((_BR, 1), lambda i: (i, 0)),
            _blk((1, _H), lambda i: (0, 0)),
        ],
        out_specs=_blk((_BR, _H), lambda i: (i, 0)),
        out_shape=jax.ShapeDtypeStruct((_N, _H), f32),
    )(A8, Z, drow, d, bg2r)

    tcol = pl.pallas_call(
        _k4,
        grid=(_NB,),
        in_specs=[
            _blk((_BR, _F), lambda i: (i, 0)),
            _blk((_N, _F), lambda i: (0, 0)),
        ],
        out_specs=_blk((_BR, 1), lambda i: (i, 0)),
        out_shape=jax.ShapeDtypeStruct((_N, 1), jnp.int32),
        scratch_shapes=[pltpu.VMEM((_BR, _N), jnp.int32)],
    )(emb, emb)

    trow = tcol.reshape(1, _N)

    new_adj, AF, d2 = pl.pallas_call(
        _k5,
        grid=(_NB, _NB),
        in_specs=[
            _blk((_BR, _F), lambda i, j: (i, 0)),
            _blk((_BR, _F), lambda i, j: (j, 0)),
            _blk((_BR, 1), lambda i, j: (i, 0)),
            _blk((1, _BR), lambda i, j: (0, j)),
            _blk((_BR, _BR), lambda i, j: (i, j)),
        ],
        out_specs=[
            _blk((_BR, _BR), lambda i, j: (i, j)),
            _blk((_BR, _BR), lambda i, j: (i, j)),
            _blk((_BR, 1), lambda i, j: (i, 0)),
        ],
        out_shape=[
            jax.ShapeDtypeStruct((_N, _N), f32),
            jax.ShapeDtypeStruct((_N, _N), f32),
            jax.ShapeDtypeStruct((_N, 1), f32),
        ],
    )(emb, emb, tcol, trow, A8)

    Z2t = pl.pallas_call(
        _k6,
        grid=(_NB,),
        in_specs=[
            _blk((_BR, _N), lambda i: (i, 0)),
            _blk((_N, _H), lambda i: (0, 0)),
            _blk((_N, 1), lambda i: (0, 0)),
            _blk((_BR, 1), lambda i: (i, 0)),
            _blk((1, _H), lambda i: (0, 0)),
            _blk((_H, _C), lambda i: (0, 0)),
        ],
        out_specs=_blk((_BR, _C), lambda i: (i, 0)),
        out_shape=jax.ShapeDtypeStruct((_N, _C), f32),
    )(AF, Pt1, d2, d2, bt1r, Wt2)

    x_out = pl.pallas_call(
        _k7,
        grid=(_NB,),
        in_specs=[
            _blk((_BR, _N), lambda i: (i, 0)),
            _blk((_N, _C), lambda i: (0, 0)),
            _blk((_BR, 1), lambda i: (i, 0)),
            _blk((1, _C), lambda i: (0, 0)),
        ],
        out_specs=_blk((_BR, _C), lambda i: (i, 0)),
        out_shape=jax.ShapeDtypeStruct((_N, _C), f32),
    )(AF, Z2t, d2, bt2r)

    return (x_out, new_adj, AF)


# speedup vs baseline: 21.8509x; 1.0093x over previous
import jax
import jax.numpy as jnp
from jax.experimental import pallas as pl
from jax.experimental.pallas import tpu as pltpu

_N = 4096
_F = 128
_H = 128
_C = 64
_K = 50
_EPS = 1e-12
_BR = 512
_NB = _N // _BR
_PREC = jax.lax.Precision.DEFAULT


def _dinv(d):
    return jnp.where(d > 0, 1.0 / jnp.sqrt(jnp.maximum(d, _EPS)), 0.0)


def _dot(a, b):
    return jnp.dot(a, b, preferred_element_type=jnp.float32, precision=_PREC)


def _sim_dot(a, b):
    hh = _F // 2
    s1 = jax.lax.dot_general(a[:, :hh], b[:, :hh], (((1,), (1,)), ((), ())),
                             preferred_element_type=jnp.float32, precision=_PREC)
    s2 = jax.lax.dot_general(a[:, hh:], b[:, hh:], (((1,), (1,)), ((), ())),
                             preferred_element_type=jnp.float32, precision=_PREC)
    return s1 + s2


def _key(x):
    b = jax.lax.bitcast_convert_type(x, jnp.int32)
    return b ^ ((b >> 31) & jnp.int32(0x7FFFFFFF))


def _k1(x_ref, adj_ref, wg1_ref, wt1_ref, d_ref, drow_ref, p1_ref, pt1_ref,
        a8_ref):
    adj = adj_ref[...]
    a8_ref[...] = adj.astype(jnp.int8)
    d_ref[...] = jnp.sum(adj, axis=1, keepdims=True)
    cs = jnp.sum(adj, axis=0, keepdims=True)

    @pl.when(pl.program_id(0) == 0)
    def _():
        drow_ref[...] = cs

    @pl.when(pl.program_id(0) != 0)
    def _():
        drow_ref[...] += cs

    xb = x_ref[...]
    p1_ref[...] = _dot(xb, wg1_ref[...])
    pt1_ref[...] = _dot(xb, wt1_ref[...])


def _na_tile(a8, db, drow):
    return (_dinv(db) * a8.astype(jnp.float32)) * _dinv(drow)


def _k2(a8_ref, p1_ref, drow_ref, db_ref, b1_ref, w2_ref, z_ref):
    na = _na_tile(a8_ref[...], db_ref[...], drow_ref[...])
    h = jnp.maximum(_dot(na, p1_ref[...]) + b1_ref[...], 0.0)
    z_ref[...] = _dot(h, w2_ref[...])


def _k3(a8_ref, z_ref, drow_ref, db_ref, b2_ref, emb_ref):
    na = _na_tile(a8_ref[...], db_ref[...], drow_ref[...])
    e = _dot(na, z_ref[...]) + b2_ref[...]
    rn = jnp.sqrt(jnp.sum(e * e, axis=1, keepdims=True))
    emb_ref[...] = e / jnp.maximum(rn, _EPS)


def _k4(embb_ref, emb_ref, tcol_ref, keys_ref):
    eb = embb_ref[...]
    for j in range(_NB):
        ch = emb_ref[j * _BR:(j + 1) * _BR, :]
        keys_ref[:, j * _BR:(j + 1) * _BR] = _key(_sim_dot(eb, ch))
    keys = keys_ref[...]
    lo = jnp.min(keys, axis=1, keepdims=True) - 1
    hi = jnp.max(keys, axis=1, keepdims=True)

    def body(_, carry):
        lo, hi = carry
        mid = lo + ((hi - lo + 1) >> 1)
        c = jnp.sum((keys > mid).astype(jnp.int32), axis=1, keepdims=True)
        ge = c >= _K
        return jnp.where(ge, mid, lo), jnp.where(ge, hi, mid)

    lo, hi = jax.lax.fori_loop(0, 31, body, (lo, hi), unroll=8)
    tcol_ref[...] = hi


def _k5(embi_ref, embj_ref, tki_ref, tkj_ref, a8_ref, na_ref, af_ref, d2_ref):
    s = _sim_dot(embi_ref[...], embj_ref[...])
    ks = _key(s)
    mr = (ks >= tki_ref[...]).astype(jnp.float32)
    mc = (ks >= tkj_ref[...]).astype(jnp.float32)
    na = 0.5 * s * (mr + mc)
    af = na + a8_ref[...].astype(jnp.float32)
    na_ref[...] = na
    af_ref[...] = af
    rs = jnp.sum(af, axis=1, keepdims=True)

    @pl.when(pl.program_id(1) == 0)
    def _():
        d2_ref[...] = rs

    @pl.when(pl.program_id(1) != 0)
    def _():
        d2_ref[...] += rs


def _k6(af_ref, pt1_ref, d2_ref, db_ref, b1_ref, w2_ref, o_ref):
    yt = _dinv(d2_ref[...]) * pt1_ref[...]
    acc = _dot(af_ref[...], yt)
    db = _dinv(db_ref[...])
    ht = jnp.maximum(db * acc + b1_ref[...], 0.0)
    o_ref[...] = db * _dot(ht, w2_ref[...])


def _k7(af_ref, z_ref, db_ref, b2_ref, o_ref):
    o_ref[...] = _dinv(db_ref[...]) * _dot(af_ref[...], z_ref[...]) + b2_ref[...]


def _blk(shape, imap):
    return pl.BlockSpec(shape, imap)


def kernel(input, Adj, Wg1, bg1, Wg2, bg2, Wt1, bt1, Wt2, bt2):
    x = input
    f32 = jnp.float32
    bg1r = bg1.reshape(1, _H)
    bg2r = bg2.reshape(1, _H)
    bt1r = bt1.reshape(1, _H)
    bt2r = bt2.reshape(1, _C)

    d, drow, P1, Pt1, A8 = pl.pallas_call(
        _k1,
        grid=(_NB,),
        in_specs=[
            _blk((_BR, _F), lambda i: (i, 0)),
            _blk((_BR, _N), lambda i: (i, 0)),
            _blk((_F, _H), lambda i: (0, 0)),
            _blk((_F, _H), lambda i: (0, 0)),
        ],
        out_specs=[
            _blk((_BR, 1), lambda i: (i, 0)),
            _blk((1, _N), lambda i: (0, 0)),
            _blk((_BR, _H), lambda i: (i, 0)),
            _blk((_BR, _H), lambda i: (i, 0)),
            _blk((_BR, _N), lambda i: (i, 0)),
        ],
        out_shape=[
            jax.ShapeDtypeStruct((_N, 1), f32),
            jax.ShapeDtypeStruct((1, _N), f32),
            jax.ShapeDtypeStruct((_N, _H), f32),
            jax.ShapeDtypeStruct((_N, _H), f32),
            jax.ShapeDtypeStruct((_N, _N), jnp.int8),
        ],
    )(x, Adj, Wg1, Wt1)

    Z = pl.pallas_call(
        _k2,
        grid=(_NB,),
        in_specs=[
            _blk((_BR, _N), lambda i: (i, 0)),
            _blk((_N, _H), lambda i: (0, 0)),
            _blk((1, _N), lambda i: (0, 0)),
            _blk((_BR, 1), lambda i: (i, 0)),
            _blk((1, _H), lambda i: (0, 0)),
            _blk((_H, _H), lambda i: (0, 0)),
        ],
        out_specs=_blk((_BR, _H), lambda i: (i, 0)),
        out_shape=jax.ShapeDtypeStruct((_N, _H), f32),
    )(A8, P1, drow, d, bg1r, Wg2)

    emb = pl.pallas_call(
        _k3,
        grid=(_NB,),
        in_specs=[
            _blk((_BR, _N), lambda i: (i, 0)),
            _blk((_N, _H), lambda i: (0, 0)),
            _blk((1, _N), lambda i: (0, 0)),
            _blk((_BR, 1), lambda i: (i, 0)),
            _blk((1, _H), lambda i: (0, 0)),
        ],
        out_specs=_blk((_BR, _H), lambda i: (i, 0)),
        out_shape=jax.ShapeDtypeStruct((_N, _H), f32),
    )(A8, Z, drow, d, bg2r)

    tcol = pl.pallas_call(
        _k4,
        grid=(_NB,),
        in_specs=[
            _blk((_BR, _F), lambda i: (i, 0)),
            _blk((_N, _F), lambda i: (0, 0)),
        ],
        out_specs=_blk((_BR, 1), lambda i: (i, 0)),
        out_shape=jax.ShapeDtypeStruct((_N, 1), jnp.int32),
        scratch_shapes=[pltpu.VMEM((_BR, _N), jnp.int32)],
    )(emb, emb)

    trow = tcol.reshape(1, _N)

    new_adj, AF, d2 = pl.pallas_call(
        _k5,
        grid=(_NB, _NB),
        in_specs=[
            _blk((_BR, _F), lambda i, j: (i, 0)),
            _blk((_BR, _F), lambda i, j: (j, 0)),
            _blk((_BR, 1), lambda i, j: (i, 0)),
            _blk((1, _BR), lambda i, j: (0, j)),
            _blk((_BR, _BR), lambda i, j: (i, j)),
        ],
        out_specs=[
            _blk((_BR, _BR), lambda i, j: (i, j)),
            _blk((_BR, _BR), lambda i, j: (i, j)),
            _blk((_BR, 1), lambda i, j: (i, 0)),
        ],
        out_shape=[
            jax.ShapeDtypeStruct((_N, _N), f32),
            jax.ShapeDtypeStruct((_N, _N), f32),
            jax.ShapeDtypeStruct((_N, 1), f32),
        ],
    )(emb, emb, tcol, trow, A8)

    Z2t = pl.pallas_call(
        _k6,
        grid=(_NB,),
        in_specs=[
            _blk((_BR, _N), lambda i: (i, 0)),
            _blk((_N, _H), lambda i: (0, 0)),
            _blk((_N, 1), lambda i: (0, 0)),
            _blk((_BR, 1), lambda i: (i, 0)),
            _blk((1, _H), lambda i: (0, 0)),
            _blk((_H, _C), lambda i: (0, 0)),
        ],
        out_specs=_blk((_BR, _C), lambda i: (i, 0)),
        out_shape=jax.ShapeDtypeStruct((_N, _C), f32),
    )(AF, Pt1, d2, d2, bt1r, Wt2)

    x_out = pl.pallas_call(
        _k7,
        grid=(_NB,),
        in_specs=[
            _blk((_BR, _N), lambda i: (i, 0)),
            _blk((_N, _C), lambda i: (0, 0)),
            _blk((_BR, 1), lambda i: (i, 0)),
            _blk((1, _C), lambda i: (0, 0)),
        ],
        out_specs=_blk((_BR, _C), lambda i: (i, 0)),
        out_shape=jax.ShapeDtypeStruct((_N, _C), f32),
    )(AF, Z2t, d2, bt2r)

    return (x_out, new_adj, AF)


# speedup vs baseline: 22.6213x; 1.0353x over previous
import jax
import jax.numpy as jnp
from jax.experimental import pallas as pl
from jax.experimental.pallas import tpu as pltpu

_N = 4096
_F = 128
_H = 128
_C = 64
_K = 50
_EPS = 1e-12
_BR = 512
_NB = _N // _BR
_PREC = jax.lax.Precision.DEFAULT


def _dinv(d):
    return jnp.where(d > 0, 1.0 / jnp.sqrt(jnp.maximum(d, _EPS)), 0.0)


def _dot(a, b):
    return jnp.dot(a, b, preferred_element_type=jnp.float32, precision=_PREC)


def _sim_dot(a, b):
    hh = _F // 2
    s1 = jax.lax.dot_general(a[:, :hh], b[:, :hh], (((1,), (1,)), ((), ())),
                             preferred_element_type=jnp.float32, precision=_PREC)
    s2 = jax.lax.dot_general(a[:, hh:], b[:, hh:], (((1,), (1,)), ((), ())),
                             preferred_element_type=jnp.float32, precision=_PREC)
    return s1 + s2


def _key(x):
    b = jax.lax.bitcast_convert_type(x, jnp.int32)
    return b ^ ((b >> 31) & jnp.int32(0x7FFFFFFF))


def _k1(x_ref, adj_ref, wg1_ref, wt1_ref, d_ref, drow_ref, p1_ref, pt1_ref,
        a8_ref):
    adj = adj_ref[...]
    a8_ref[...] = adj.astype(jnp.int8)
    d_ref[...] = jnp.sum(adj, axis=1, keepdims=True)
    cs = jnp.sum(adj, axis=0, keepdims=True)

    @pl.when(pl.program_id(0) == 0)
    def _():
        drow_ref[...] = cs

    @pl.when(pl.program_id(0) != 0)
    def _():
        drow_ref[...] += cs

    xb = x_ref[...]
    p1_ref[...] = _dot(xb, wg1_ref[...])
    pt1_ref[...] = _dot(xb, wt1_ref[...])


def _na_tile(a8, db, drow):
    return (_dinv(db) * a8.astype(jnp.float32)) * _dinv(drow)


def _k2(a8_ref, p1_ref, drow_ref, db_ref, b1_ref, w2_ref, z_ref):
    na = _na_tile(a8_ref[...], db_ref[...], drow_ref[...])
    h = jnp.maximum(_dot(na, p1_ref[...]) + b1_ref[...], 0.0)
    z_ref[...] = _dot(h, w2_ref[...])


def _k3(a8_ref, z_ref, drow_ref, db_ref, b2_ref, emb_ref):
    na = _na_tile(a8_ref[...], db_ref[...], drow_ref[...])
    e = _dot(na, z_ref[...]) + b2_ref[...]
    rn = jnp.sqrt(jnp.sum(e * e, axis=1, keepdims=True))
    emb_ref[...] = e / jnp.maximum(rn, _EPS)


def _inv_key(k):
    b = k ^ ((k >> 31) & jnp.int32(0x7FFFFFFF))
    return jax.lax.bitcast_convert_type(b, jnp.float32)


def _step_decode(s):
    m = jnp.floor(jnp.sqrt(s.astype(jnp.float32) + 0.5)).astype(jnp.int32)
    r = s - m * m
    ti = jnp.where((r >= 1) & (r <= m), r - 1, m)
    tj = jnp.where(r <= m, m, r - m - 1)
    return m, r, ti, tj


def _k45(emb_ref, a8_ref, na_ref, af_ref, d2_ref, keys_ref, tcf_ref, trf_ref,
         acc_ref):
    s_id = pl.program_id(0)
    m, r, ti, tj = _step_decode(s_id)

    @pl.when(s_id == 0)
    def _():
        acc_ref[...] = jnp.zeros((_N, 1), jnp.float32)

    @pl.when(r == 0)
    def _():
        eb = emb_ref[pl.ds(m * _BR, _BR), :]
        for j8 in range(_NB):
            ch = emb_ref[j8 * _BR:(j8 + 1) * _BR, :]
            keys_ref[:, j8 * _BR:(j8 + 1) * _BR] = _key(_sim_dot(eb, ch))
        keys = keys_ref[...]
        lo = jnp.min(keys, axis=1, keepdims=True) - 1
        hi = jnp.max(keys, axis=1, keepdims=True)

        def body(_, carry):
            lo, hi = carry
            mid = lo + ((hi - lo + 1) >> 1)
            c = jnp.sum((keys > mid).astype(jnp.int32), axis=1, keepdims=True)
            ge = c >= _K
            return jnp.where(ge, mid, lo), jnp.where(ge, hi, mid)

        lo, hi = jax.lax.fori_loop(0, 31, body, (lo, hi), unroll=8)
        tf = _inv_key(hi)
        tcf_ref[pl.ds(m * _BR, _BR), :] = tf
        trf_ref[0, pl.ds(m * _BR, _BR)] = jnp.swapaxes(tf, 0, 1)[0, :]

    is_col = (r >= 1) & (r <= m)
    cidx = jnp.where(is_col, ti, tj)
    sraw = _inv_key(keys_ref[:, pl.ds(cidx * _BR, _BR)])
    stile = jnp.where(is_col, jnp.swapaxes(sraw, 0, 1), sraw)
    tc = tcf_ref[pl.ds(ti * _BR, _BR), :]
    tr = trf_ref[0:1, pl.ds(tj * _BR, _BR)]
    mr = (stile >= tc).astype(jnp.float32)
    mc = (stile >= tr).astype(jnp.float32)
    na = 0.5 * stile * (mr + mc)
    af = na + a8_ref[...].astype(jnp.float32)
    na_ref[...] = na
    af_ref[...] = af
    acc_ref[pl.ds(ti * _BR, _BR), :] += jnp.sum(af, axis=1, keepdims=True)

    @pl.when(s_id == _NB * _NB - 1)
    def _():
        d2_ref[...] = acc_ref[...]


def _k6(af_ref, pt1_ref, d2_ref, db_ref, b1_ref, w2_ref, o_ref):
    yt = _dinv(d2_ref[...]) * pt1_ref[...]
    acc = _dot(af_ref[...], yt)
    db = _dinv(db_ref[...])
    ht = jnp.maximum(db * acc + b1_ref[...], 0.0)
    o_ref[...] = db * _dot(ht, w2_ref[...])


def _k7(af_ref, z_ref, db_ref, b2_ref, o_ref):
    o_ref[...] = _dinv(db_ref[...]) * _dot(af_ref[...], z_ref[...]) + b2_ref[...]


def _blk(shape, imap):
    return pl.BlockSpec(shape, imap)


def kernel(input, Adj, Wg1, bg1, Wg2, bg2, Wt1, bt1, Wt2, bt2):
    x = input
    f32 = jnp.float32
    bg1r = bg1.reshape(1, _H)
    bg2r = bg2.reshape(1, _H)
    bt1r = bt1.reshape(1, _H)
    bt2r = bt2.reshape(1, _C)

    d, drow, P1, Pt1, A8 = pl.pallas_call(
        _k1,
        grid=(_NB,),
        in_specs=[
            _blk((_BR, _F), lambda i: (i, 0)),
            _blk((_BR, _N), lambda i: (i, 0)),
            _blk((_F, _H), lambda i: (0, 0)),
            _blk((_F, _H), lambda i: (0, 0)),
        ],
        out_specs=[
            _blk((_BR, 1), lambda i: (i, 0)),
            _blk((1, _N), lambda i: (0, 0)),
            _blk((_BR, _H), lambda i: (i, 0)),
            _blk((_BR, _H), lambda i: (i, 0)),
            _blk((_BR, _N), lambda i: (i, 0)),
        ],
        out_shape=[
            jax.ShapeDtypeStruct((_N, 1), f32),
            jax.ShapeDtypeStruct((1, _N), f32),
            jax.ShapeDtypeStruct((_N, _H), f32),
            jax.ShapeDtypeStruct((_N, _H), f32),
            jax.ShapeDtypeStruct((_N, _N), jnp.int8),
        ],
    )(x, Adj, Wg1, Wt1)

    Z = pl.pallas_call(
        _k2,
        grid=(_NB,),
        in_specs=[
            _blk((_BR, _N), lambda i: (i, 0)),
            _blk((_N, _H), lambda i: (0, 0)),
            _blk((1, _N), lambda i: (0, 0)),
            _blk((_BR, 1), lambda i: (i, 0)),
            _blk((1, _H), lambda i: (0, 0)),
            _blk((_H, _H), lambda i: (0, 0)),
        ],
        out_specs=_blk((_BR, _H), lambda i: (i, 0)),
        out_shape=jax.ShapeDtypeStruct((_N, _H), f32),
    )(A8, P1, drow, d, bg1r, Wg2)

    emb = pl.pallas_call(
        _k3,
        grid=(_NB,),
        in_specs=[
            _blk((_BR, _N), lambda i: (i, 0)),
            _blk((_N, _H), lambda i: (0, 0)),
            _blk((1, _N), lambda i: (0, 0)),
            _blk((_BR, 1), lambda i: (i, 0)),
            _blk((1, _H), lambda i: (0, 0)),
        ],
        out_specs=_blk((_BR, _H), lambda i: (i, 0)),
        out_shape=jax.ShapeDtypeStruct((_N, _H), f32),
    )(A8, Z, drow, d, bg2r)

    def _ti(s):
        return _step_decode(s)[2]

    def _tj(s):
        return _step_decode(s)[3]

    new_adj, AF, d2 = pl.pallas_call(
        _k45,
        grid=(_NB * _NB,),
        in_specs=[
            _blk((_N, _F), lambda s: (0, 0)),
            _blk((_BR, _BR), lambda s: (_ti(s), _tj(s))),
        ],
        out_specs=[
            _blk((_BR, _BR), lambda s: (_ti(s), _tj(s))),
            _blk((_BR, _BR), lambda s: (_ti(s), _tj(s))),
            _blk((_N, 1), lambda s: (0, 0)),
        ],
        out_shape=[
            jax.ShapeDtypeStruct((_N, _N), f32),
            jax.ShapeDtypeStruct((_N, _N), f32),
            jax.ShapeDtypeStruct((_N, 1), f32),
        ],
        scratch_shapes=[
            pltpu.VMEM((_BR, _N), jnp.int32),
            pltpu.VMEM((_N, 1), jnp.float32),
            pltpu.VMEM((1, _N), jnp.float32),
            pltpu.VMEM((_N, 1), jnp.float32),
        ],
    )(emb, A8)

    Z2t = pl.pallas_call(
        _k6,
        grid=(_NB,),
        in_specs=[
            _blk((_BR, _N), lambda i: (i, 0)),
            _blk((_N, _H), lambda i: (0, 0)),
            _blk((_N, 1), lambda i: (0, 0)),
            _blk((_BR, 1), lambda i: (i, 0)),
            _blk((1, _H), lambda i: (0, 0)),
            _blk((_H, _C), lambda i: (0, 0)),
        ],
        out_specs=_blk((_BR, _C), lambda i: (i, 0)),
        out_shape=jax.ShapeDtypeStruct((_N, _C), f32),
    )(AF, Pt1, d2, d2, bt1r, Wt2)

    x_out = pl.pallas_call(
        _k7,
        grid=(_NB,),
        in_specs=[
            _blk((_BR, _N), lambda i: (i, 0)),
            _blk((_N, _C), lambda i: (0, 0)),
            _blk((_BR, 1), lambda i: (i, 0)),
            _blk((1, _C), lambda i: (0, 0)),
        ],
        out_specs=_blk((_BR, _C), lambda i: (i, 0)),
        out_shape=jax.ShapeDtypeStruct((_N, _C), f32),
    )(AF, Z2t, d2, bt2r)

    return (x_out, new_adj, AF)
